# Initial kernel scaffold; baseline (speedup 1.0000x reference)
import jax
import jax.numpy as jnp
from jax.experimental import pallas as pl


def kernel(x, W, b):
    raise NotImplementedError("write your pallas kernel here")


# speedup vs baseline: 1.0320x; 1.0320x over previous
import jax
import jax.numpy as jnp
from jax.experimental import pallas as pl

_TOPK = 8


def _router_body(x_ref, w_ref, b_ref, gates_ref, idx_ref):
    x = x_ref[...]
    w = w_ref[...]
    logits = jax.lax.dot_general(
        x, w, (((1,), (1,)), ((), ())), preferred_element_type=jnp.float32
    )
    logits = logits + b_ref[...]
    e = logits.shape[1]
    iota = jax.lax.broadcasted_iota(jnp.int32, logits.shape, 1)
    vals, idxs = [], []
    l = logits
    for _ in range(_TOPK):
        m = jnp.max(l, axis=1, keepdims=True)
        is_max = l >= m
        cand = jnp.min(jnp.where(is_max, iota, e), axis=1, keepdims=True)
        vals.append(m)
        idxs.append(cand)
        l = jnp.where(iota == cand, -jnp.inf, l)
    v = jnp.concatenate(vals, axis=1)
    ix = jnp.concatenate(idxs, axis=1)
    ex = jnp.exp(v - v[:, 0:1])
    gates_ref[...] = ex / jnp.sum(ex, axis=1, keepdims=True)
    idx_ref[...] = ix


def kernel(x, W, b):
    n, d = x.shape
    e = W.shape[0]
    tile = 512 if n % 512 == 0 else n
    gates, idx = pl.pallas_call(
        _router_body,
        grid=(n // tile,),
        in_specs=[
            pl.BlockSpec((tile, d), lambda i: (i, 0)),
            pl.BlockSpec((e, d), lambda i: (0, 0)),
            pl.BlockSpec((1, e), lambda i: (0, 0)),
        ],
        out_specs=[
            pl.BlockSpec((tile, _TOPK), lambda i: (i, 0)),
            pl.BlockSpec((tile, _TOPK), lambda i: (i, 0)),
        ],
        out_shape=[
            jax.ShapeDtypeStruct((n, _TOPK), jnp.float32),
            jax.ShapeDtypeStruct((n, _TOPK), jnp.int32),
        ],
    )(x, W, b.reshape(1, e))
    return gates, idx.astype(jnp.int64)


# speedup vs baseline: 1.1253x; 1.0903x over previous
import jax
import jax.numpy as jnp
from jax.experimental import pallas as pl

_TOPK = 8


def _router_body(x_ref, w_ref, b_ref, gates_ref, idx_ref):
    x = x_ref[...]
    w = w_ref[...]
    logits = jax.lax.dot_general(
        x, w, (((1,), (1,)), ((), ())), preferred_element_type=jnp.float32
    )
    logits = logits + b_ref[...]
    e = logits.shape[1]
    iota = jax.lax.broadcasted_iota(jnp.int32, logits.shape, 1)
    vals, idxs = [], []
    l = logits
    for _ in range(_TOPK):
        m = jnp.max(l, axis=1, keepdims=True)
        is_max = l >= m
        cand = jnp.min(jnp.where(is_max, iota, e), axis=1, keepdims=True)
        vals.append(m)
        idxs.append(cand)
        l = jnp.where(iota == cand, -jnp.inf, l)
    v = jnp.concatenate(vals, axis=1)
    ix = jnp.concatenate(idxs, axis=1)
    ex = jnp.exp(v - v[:, 0:1])
    gates_ref[...] = ex / jnp.sum(ex, axis=1, keepdims=True)
    idx_ref[...] = ix


def kernel(x, W, b):
    n, d = x.shape
    e = W.shape[0]
    tile = 1024 if n % 1024 == 0 else n
    gates, idx = pl.pallas_call(
        _router_body,
        grid=(n // tile,),
        in_specs=[
            pl.BlockSpec((tile, d), lambda i: (i, 0)),
            pl.BlockSpec((e, d), lambda i: (0, 0)),
            pl.BlockSpec((1, e), lambda i: (0, 0)),
        ],
        out_specs=[
            pl.BlockSpec((tile, _TOPK), lambda i: (i, 0)),
            pl.BlockSpec((tile, _TOPK), lambda i: (i, 0)),
        ],
        out_shape=[
            jax.ShapeDtypeStruct((n, _TOPK), jnp.float32),
            jax.ShapeDtypeStruct((n, _TOPK), jnp.int32),
        ],
    )(x, W, b.reshape(1, e))
    return gates, idx.astype(jnp.int64)


# speedup vs baseline: 1.4603x; 1.2978x over previous
import functools

import jax
import jax.numpy as jnp
from jax import lax
from jax.experimental import pallas as pl
from jax.experimental.pallas import tpu as pltpu
from jax.experimental.pallas import tpu_sc as plsc

_TOPK = 8

_SORT8 = [(0, 1), (2, 3), (4, 5), (6, 7),
          (0, 2), (1, 3), (4, 6), (5, 7),
          (1, 2), (5, 6),
          (0, 4), (1, 5), (2, 6), (3, 7),
          (1, 4), (3, 6),
          (2, 4), (3, 5),
          (3, 4)]
_BMERGE8 = [(0, 4), (1, 5), (2, 6), (3, 7),
            (0, 2), (1, 3), (4, 6), (5, 7),
            (0, 1), (2, 3), (4, 5), (6, 7)]


def _cas(p, q):
    pv, pi = p
    qv, qi = q
    c = pv >= qv
    hi = (jnp.where(c, pv, qv), jnp.where(c, pi, qi))
    lo = (jnp.where(c, qv, pv), jnp.where(c, qi, pi))
    return hi, lo


def _sort8(pairs):
    pairs = list(pairs)
    for a, b in _SORT8:
        pairs[a], pairs[b] = _cas(pairs[a], pairs[b])
    return pairs


def _merge_top8(A, B):
    c = []
    for i in range(8):
        pv, pi = A[i]
        qv, qi = B[7 - i]
        m = pv >= qv
        c.append((jnp.where(m, pv, qv), jnp.where(m, pi, qi)))
    for a, b in _BMERGE8:
        c[a], c[b] = _cas(c[a], c[b])
    return c


def _matmul_body(x_ref, w_ref, b_ref, lt_ref):
    lt = jax.lax.dot_general(
        w_ref[...], x_ref[...], (((1,), (1,)), ((), ())),
        preferred_element_type=jnp.float32,
    )
    lt_ref[...] = lt + b_ref[...]


def _logits_t(x, W, b, tile):
    n, d = x.shape
    e = W.shape[0]
    return pl.pallas_call(
        _matmul_body,
        grid=(n // tile,),
        in_specs=[
            pl.BlockSpec((tile, d), lambda i: (i, 0)),
            pl.BlockSpec((e, d), lambda i: (0, 0)),
            pl.BlockSpec((e, 1), lambda i: (0, 0)),
        ],
        out_specs=pl.BlockSpec((e, tile), lambda i: (0, i)),
        out_shape=jax.ShapeDtypeStruct((e, n), jnp.float32),
    )(x, W, b.reshape(e, 1))


def _make_sc_topk(n, e):
    info = plsc.get_sparse_core_info()
    nc, ns, nl = info.num_cores, info.num_subcores, info.num_lanes
    nw = nc * ns
    assert n % (nw * nl) == 0 and e == 64
    tok_w = n // nw
    ngroups = tok_w // nl
    mesh = plsc.VectorSubcoreMesh(core_axis_name="c", subcore_axis_name="s")

    @functools.partial(
        pl.kernel, mesh=mesh,
        out_type=[
            jax.ShapeDtypeStruct((_TOPK, n), jnp.float32),
            jax.ShapeDtypeStruct((_TOPK, n), jnp.int32),
        ],
        scratch_types=[
            pltpu.VMEM((e, tok_w), jnp.float32),
            pltpu.VMEM((_TOPK, tok_w), jnp.float32),
            pltpu.VMEM((_TOPK, tok_w), jnp.int32),
        ],
    )
    def sc_topk(lt_hbm, gt_hbm, it_hbm, lt_v, g_v, i_v):
        wid = lax.axis_index("s") * nc + lax.axis_index("c")
        base = wid * tok_w
        pltpu.sync_copy(lt_hbm.at[:, pl.ds(base, tok_w)], lt_v)

        def group_body(g, carry):
            off = g * nl

            def sorted_group(j):
                pairs = [
                    (lt_v[8 * j + t, pl.ds(off, nl)],
                     jnp.full((nl,), 8 * j + t, jnp.int32))
                    for t in range(8)
                ]
                return _sort8(pairs)

            top = sorted_group(0)
            for j in range(1, 8):
                top = _merge_top8(top, sorted_group(j))

            m = top[0][0]
            exps = [jnp.exp(tv - m) for tv, _ in top]
            denom = exps[0]
            for s in exps[1:]:
                denom = denom + s
            inv = 1.0 / denom
            for k in range(_TOPK):
                g_v[k, pl.ds(off, nl)] = exps[k] * inv
                i_v[k, pl.ds(off, nl)] = top[k][1]
            return carry

        lax.fori_loop(0, ngroups, group_body, 0)
        pltpu.sync_copy(g_v, gt_hbm.at[:, pl.ds(base, tok_w)])
        pltpu.sync_copy(i_v, it_hbm.at[:, pl.ds(base, tok_w)])

    return sc_topk


def kernel(x, W, b):
    n, d = x.shape
    e = W.shape[0]
    tile = 1024 if n % 1024 == 0 else n
    lt = _logits_t(x, W, b, tile)
    gt, it = _make_sc_topk(n, e)(lt)
    return gt.T, it.T.astype(jnp.int64)
